# baseline (device time: 33275 ns/iter reference)
import jax
import jax.numpy as jnp
from jax import lax
from jax.experimental import pallas as pl
from jax.experimental.pallas import tpu as pltpu

N_DEV = 4


def kernel(ids, E):
    v_per, d = E.shape
    t = ids.shape[0]

    my_pos = lax.axis_index("i")
    offset = my_pos * v_per
    local = ids - offset
    in_range = (local >= 0) & (local < v_per)
    safe = jnp.where(in_range, local, 0)
    partial = jnp.take(E, safe, axis=0) * in_range[:, None].astype(E.dtype)
    return _ring_all_reduce(partial)


def _ring_all_reduce(x):
    t, d = x.shape
    chunk = t // N_DEV

    def body(x_ref, out_ref, comm_ref,
             rs_send_sems, rs_recv_sems, ag_send_sems, ag_recv_sems):
        my = lax.axis_index("i")
        left = lax.rem(my + (N_DEV - 1), N_DEV)
        right = lax.rem(my + 1, N_DEV)

        barrier_sem = pltpu.get_barrier_semaphore()
        for nbr in (left, right):
            pl.semaphore_signal(
                barrier_sem, inc=1,
                device_id=(nbr,), device_id_type=pl.DeviceIdType.MESH,
            )
        pl.semaphore_wait(barrier_sem, 2)

        out_ref[...] = x_ref[...]

        for h in range(N_DEV - 1):
            send_idx = lax.rem(my - h + 2 * N_DEV, N_DEV)
            recv_idx = lax.rem(my - h - 1 + 2 * N_DEV, N_DEV)
            rdma = pltpu.make_async_remote_copy(
                src_ref=out_ref.at[pl.ds(send_idx * chunk, chunk), :],
                dst_ref=comm_ref.at[h],
                send_sem=rs_send_sems.at[h],
                recv_sem=rs_recv_sems.at[h],
                device_id=(right,),
                device_id_type=pl.DeviceIdType.MESH,
            )
            rdma.start()
            rdma.wait()
            acc = out_ref[pl.ds(recv_idx * chunk, chunk), :] + comm_ref[h]
            out_ref[pl.ds(recv_idx * chunk, chunk), :] = acc

        for g in range(N_DEV - 1):
            send_idx = lax.rem(my + 1 - g + 2 * N_DEV, N_DEV)
            recv_idx = lax.rem(my - g + 2 * N_DEV, N_DEV)
            rdma = pltpu.make_async_remote_copy(
                src_ref=out_ref.at[pl.ds(send_idx * chunk, chunk), :],
                dst_ref=out_ref.at[pl.ds(send_idx * chunk, chunk), :],
                send_sem=ag_send_sems.at[g],
                recv_sem=ag_recv_sems.at[g],
                device_id=(right,),
                device_id_type=pl.DeviceIdType.MESH,
            )
            rdma.start()
            rdma.wait()

    return pl.pallas_call(
        body,
        out_shape=jax.ShapeDtypeStruct((t, d), x.dtype),
        in_specs=[pl.BlockSpec(memory_space=pltpu.VMEM)],
        out_specs=pl.BlockSpec(memory_space=pltpu.VMEM),
        scratch_shapes=[
            pltpu.VMEM((N_DEV - 1, chunk, d), x.dtype),
            pltpu.SemaphoreType.DMA((N_DEV - 1,)),
            pltpu.SemaphoreType.DMA((N_DEV - 1,)),
            pltpu.SemaphoreType.DMA((N_DEV - 1,)),
            pltpu.SemaphoreType.DMA((N_DEV - 1,)),
        ],
        compiler_params=pltpu.CompilerParams(collective_id=0),
    )(x)


# device time: 21795 ns/iter; 1.5267x vs baseline; 1.5267x over previous
import jax
import jax.numpy as jnp
from jax import lax
from jax.experimental import pallas as pl
from jax.experimental.pallas import tpu as pltpu

N_DEV = 4


def kernel(ids, E):
    v_per, d = E.shape

    my_pos = lax.axis_index("i")
    offset = my_pos * v_per
    local = ids - offset
    in_range = (local >= 0) & (local < v_per)
    safe = jnp.where(in_range, local, 0)
    partial = jnp.take(E, safe, axis=0) * in_range[:, None].astype(E.dtype)
    return _direct_all_reduce(partial)


def _direct_all_reduce(x):
    t, d = x.shape
    chunk = t // N_DEV

    def body(x_ref, out_ref, rs_buf,
             rs_send_sems, rs_recv_sems, ag_send_sems, ag_recv_sems):
        my = lax.axis_index("i")

        barrier_sem = pltpu.get_barrier_semaphore()
        for k in range(1, N_DEV):
            peer = lax.rem(my + k, N_DEV)
            pl.semaphore_signal(
                barrier_sem, inc=1,
                device_id=(peer,), device_id_type=pl.DeviceIdType.MESH,
            )
        pl.semaphore_wait(barrier_sem, N_DEV - 1)

        rs = []
        for k in range(1, N_DEV):
            peer = lax.rem(my + k, N_DEV)
            r = pltpu.make_async_remote_copy(
                src_ref=x_ref.at[pl.ds(peer * chunk, chunk), :],
                dst_ref=rs_buf.at[k - 1],
                send_sem=rs_send_sems.at[k - 1],
                recv_sem=rs_recv_sems.at[k - 1],
                device_id=(peer,),
                device_id_type=pl.DeviceIdType.MESH,
            )
            r.start()
            rs.append(r)
        for r in rs:
            r.wait_recv()
        acc = (x_ref[pl.ds(my * chunk, chunk), :]
               + rs_buf[0] + rs_buf[1] + rs_buf[2])
        out_ref[pl.ds(my * chunk, chunk), :] = acc

        ag = []
        for k in range(1, N_DEV):
            peer = lax.rem(my + k, N_DEV)
            r = pltpu.make_async_remote_copy(
                src_ref=out_ref.at[pl.ds(my * chunk, chunk), :],
                dst_ref=out_ref.at[pl.ds(my * chunk, chunk), :],
                send_sem=ag_send_sems.at[k - 1],
                recv_sem=ag_recv_sems.at[k - 1],
                device_id=(peer,),
                device_id_type=pl.DeviceIdType.MESH,
            )
            r.start()
            ag.append(r)
        for r in rs:
            r.wait_send()
        for r in ag:
            r.wait_recv()
        for r in ag:
            r.wait_send()

    return pl.pallas_call(
        body,
        out_shape=jax.ShapeDtypeStruct((t, d), x.dtype),
        in_specs=[pl.BlockSpec(memory_space=pltpu.VMEM)],
        out_specs=pl.BlockSpec(memory_space=pltpu.VMEM),
        scratch_shapes=[
            pltpu.VMEM((N_DEV - 1, chunk, d), x.dtype),
            pltpu.SemaphoreType.DMA((N_DEV - 1,)),
            pltpu.SemaphoreType.DMA((N_DEV - 1,)),
            pltpu.SemaphoreType.DMA((N_DEV - 1,)),
            pltpu.SemaphoreType.DMA((N_DEV - 1,)),
        ],
        compiler_params=pltpu.CompilerParams(collective_id=0),
    )(x)


# device time: 16240 ns/iter; 2.0490x vs baseline; 1.3421x over previous
import jax
import jax.numpy as jnp
from jax import lax
from jax.experimental import pallas as pl
from jax.experimental.pallas import tpu as pltpu

N_DEV = 4


def kernel(ids, E):
    v_per, d = E.shape

    my_pos = lax.axis_index("i")
    offset = my_pos * v_per
    local = ids - offset
    in_range = (local >= 0) & (local < v_per)
    safe = jnp.where(in_range, local, 0)
    partial = jnp.take(E, safe, axis=0) * in_range[:, None].astype(E.dtype)
    return _direct_all_reduce(partial.astype(jnp.bfloat16))


def _direct_all_reduce(xb):
    t, d = xb.shape
    chunk = t // N_DEV

    def body(x_ref, out_ref, rs_buf, ag_buf, red_bf,
             rs_send_sems, rs_recv_sems, ag_send_sems, ag_recv_sems):
        my = lax.axis_index("i")

        barrier_sem = pltpu.get_barrier_semaphore()
        for k in range(1, N_DEV):
            peer = lax.rem(my + k, N_DEV)
            pl.semaphore_signal(
                barrier_sem, inc=1,
                device_id=(peer,), device_id_type=pl.DeviceIdType.MESH,
            )
        pl.semaphore_wait(barrier_sem, N_DEV - 1)

        rs = []
        for k in range(1, N_DEV):
            peer = lax.rem(my + k, N_DEV)
            r = pltpu.make_async_remote_copy(
                src_ref=x_ref.at[pl.ds(peer * chunk, chunk), :],
                dst_ref=rs_buf.at[k - 1],
                send_sem=rs_send_sems.at[k - 1],
                recv_sem=rs_recv_sems.at[k - 1],
                device_id=(peer,),
                device_id_type=pl.DeviceIdType.MESH,
            )
            r.start()
            rs.append(r)
        for r in rs:
            r.wait_recv()
        acc = (x_ref[pl.ds(my * chunk, chunk), :].astype(jnp.float32)
               + rs_buf[0].astype(jnp.float32)
               + rs_buf[1].astype(jnp.float32)
               + rs_buf[2].astype(jnp.float32))
        out_ref[pl.ds(my * chunk, chunk), :] = acc
        red_bf[...] = acc.astype(jnp.bfloat16)

        ag = []
        for k in range(1, N_DEV):
            peer = lax.rem(my + k, N_DEV)
            r = pltpu.make_async_remote_copy(
                src_ref=red_bf,
                dst_ref=ag_buf.at[k - 1],
                send_sem=ag_send_sems.at[k - 1],
                recv_sem=ag_recv_sems.at[k - 1],
                device_id=(peer,),
                device_id_type=pl.DeviceIdType.MESH,
            )
            r.start()
            ag.append(r)
        for r in rs:
            r.wait_send()
        for k in range(1, N_DEV):
            src = lax.rem(my - k + 2 * N_DEV, N_DEV)
            ag[k - 1].wait_recv()
            out_ref[pl.ds(src * chunk, chunk), :] = (
                ag_buf[k - 1].astype(jnp.float32))
        for r in ag:
            r.wait_send()

    return pl.pallas_call(
        body,
        out_shape=jax.ShapeDtypeStruct((t, d), jnp.float32),
        in_specs=[pl.BlockSpec(memory_space=pltpu.VMEM)],
        out_specs=pl.BlockSpec(memory_space=pltpu.VMEM),
        scratch_shapes=[
            pltpu.VMEM((N_DEV - 1, chunk, d), jnp.bfloat16),
            pltpu.VMEM((N_DEV - 1, chunk, d), jnp.bfloat16),
            pltpu.VMEM((chunk, d), jnp.bfloat16),
            pltpu.SemaphoreType.DMA((N_DEV - 1,)),
            pltpu.SemaphoreType.DMA((N_DEV - 1,)),
            pltpu.SemaphoreType.DMA((N_DEV - 1,)),
            pltpu.SemaphoreType.DMA((N_DEV - 1,)),
        ],
        compiler_params=pltpu.CompilerParams(collective_id=0),
    )(xb)


# device time: 15068 ns/iter; 2.2083x vs baseline; 1.0778x over previous
import jax
import jax.numpy as jnp
from jax import lax
from jax.experimental import pallas as pl
from jax.experimental.pallas import tpu as pltpu

N_DEV = 4


def kernel(ids, E):
    v_per, d = E.shape

    my_pos = lax.axis_index("i")
    offset = my_pos * v_per
    local = ids - offset
    in_range = (local >= 0) & (local < v_per)
    safe = jnp.where(in_range, local, 0)
    partial = jnp.take(E, safe, axis=0) * in_range[:, None].astype(E.dtype)
    return _direct_all_reduce(partial.astype(jnp.bfloat16))


N_HALF = 2


def _direct_all_reduce(xb):
    t, d = xb.shape
    chunk = t // N_DEV
    dh = d // N_HALF

    def body(x_ref, out_ref, rs_buf, ag_buf, red_bf,
             rs_send_sems, rs_recv_sems, ag_send_sems, ag_recv_sems):
        my = lax.axis_index("i")

        barrier_sem = pltpu.get_barrier_semaphore()
        for k in range(1, N_DEV):
            peer = lax.rem(my + k, N_DEV)
            pl.semaphore_signal(
                barrier_sem, inc=1,
                device_id=(peer,), device_id_type=pl.DeviceIdType.MESH,
            )
        pl.semaphore_wait(barrier_sem, N_DEV - 1)

        rs = [[None] * (N_DEV - 1) for _ in range(N_HALF)]
        for h in range(N_HALF):
            cs = pl.ds(h * dh, dh)
            for k in range(1, N_DEV):
                peer = lax.rem(my + k, N_DEV)
                r = pltpu.make_async_remote_copy(
                    src_ref=x_ref.at[pl.ds(peer * chunk, chunk), cs],
                    dst_ref=rs_buf.at[k - 1, :, cs],
                    send_sem=rs_send_sems.at[h, k - 1],
                    recv_sem=rs_recv_sems.at[h, k - 1],
                    device_id=(peer,),
                    device_id_type=pl.DeviceIdType.MESH,
                )
                r.start()
                rs[h][k - 1] = r
        ag = [[None] * (N_DEV - 1) for _ in range(N_HALF)]
        for h in range(N_HALF):
            cs = pl.ds(h * dh, dh)
            for r in rs[h]:
                r.wait_recv()
            acc = (x_ref[pl.ds(my * chunk, chunk), cs].astype(jnp.float32)
                   + rs_buf[0, :, cs].astype(jnp.float32)
                   + rs_buf[1, :, cs].astype(jnp.float32)
                   + rs_buf[2, :, cs].astype(jnp.float32))
            out_ref[pl.ds(my * chunk, chunk), cs] = acc
            red_bf[:, cs] = acc.astype(jnp.bfloat16)
            for k in range(1, N_DEV):
                peer = lax.rem(my + k, N_DEV)
                r = pltpu.make_async_remote_copy(
                    src_ref=red_bf.at[:, cs],
                    dst_ref=ag_buf.at[k - 1, :, cs],
                    send_sem=ag_send_sems.at[h, k - 1],
                    recv_sem=ag_recv_sems.at[h, k - 1],
                    device_id=(peer,),
                    device_id_type=pl.DeviceIdType.MESH,
                )
                r.start()
                ag[h][k - 1] = r
        for h in range(N_HALF):
            cs = pl.ds(h * dh, dh)
            for k in range(1, N_DEV):
                src = lax.rem(my - k + 2 * N_DEV, N_DEV)
                ag[h][k - 1].wait_recv()
                out_ref[pl.ds(src * chunk, chunk), cs] = (
                    ag_buf[k - 1, :, cs].astype(jnp.float32))
        for h in range(N_HALF):
            for r in rs[h]:
                r.wait_send()
            for r in ag[h]:
                r.wait_send()

    return pl.pallas_call(
        body,
        out_shape=jax.ShapeDtypeStruct((t, d), jnp.float32),
        in_specs=[pl.BlockSpec(memory_space=pltpu.VMEM)],
        out_specs=pl.BlockSpec(memory_space=pltpu.VMEM),
        scratch_shapes=[
            pltpu.VMEM((N_DEV - 1, chunk, d), jnp.bfloat16),
            pltpu.VMEM((N_DEV - 1, chunk, d), jnp.bfloat16),
            pltpu.VMEM((chunk, d), jnp.bfloat16),
            pltpu.SemaphoreType.DMA((N_HALF, N_DEV - 1)),
            pltpu.SemaphoreType.DMA((N_HALF, N_DEV - 1)),
            pltpu.SemaphoreType.DMA((N_HALF, N_DEV - 1)),
            pltpu.SemaphoreType.DMA((N_HALF, N_DEV - 1)),
        ],
        compiler_params=pltpu.CompilerParams(collective_id=0),
    )(xb)
